# Initial kernel scaffold; baseline (speedup 1.0000x reference)
import jax
import jax.numpy as jnp
from jax.experimental import pallas as pl


def kernel(x, edge_index, etypes, weight, w_comp, h_bias, loop_weight):
    raise NotImplementedError("write your pallas kernel here")


# speedup vs baseline: 3.6217x; 3.6217x over previous
import jax
import jax.numpy as jnp
from jax import lax
from jax.experimental import pallas as pl
from jax.experimental.pallas import tpu as pltpu
from jax.experimental.pallas import tpu_sc as plsc

N = 10000
E = 320000
IN_FEAT = 128
OUT_FEAT = 128
NUM_RELS = 32
NUM_BASES = 8

NC = 2
NS = 16
NW = NC * NS
CHUNK = 128
NCHUNKS = -(-E // (CHUNK * NW)) * NW
E_PAD = NCHUNKS * CHUNK
CPW = NCHUNKS // NW
NPAD = 10240
ROWS_PER_TILE = NPAD // NS
BN = 1000


def _xw_body(w_comp_ref, weight_ref, x_ref, out_ref):
    r = pl.program_id(1)
    w = w_comp_ref[r, 0] * weight_ref[0]
    for b in range(1, NUM_BASES):
        w = w + w_comp_ref[r, b] * weight_ref[b]
    out_ref[0] = jnp.dot(x_ref[...], w, preferred_element_type=jnp.float32)


def _sc_body(xw_hbm, idx_hbm, dst_hbm, zeros_hbm, out_hbm,
             idx_v, dst_v, rows_v, h_shared, sem):
    c = lax.axis_index("c")
    s = lax.axis_index("s")
    wid = s * NC + c
    tile_base = s * ROWS_PER_TILE
    pltpu.sync_copy(zeros_hbm, h_shared.at[pl.ds(tile_base, ROWS_PER_TILE)])
    plsc.subcore_barrier()

    def step(j, carry):
        g = wid * CPW + j
        pltpu.sync_copy(idx_hbm.at[g], idx_v)
        pltpu.sync_copy(dst_hbm.at[g], dst_v)
        pltpu.async_copy(xw_hbm.at[idx_v], rows_v, sem).wait()
        pltpu.sync_copy(rows_v, h_shared.at[dst_v], add=True)
        return carry

    lax.fori_loop(0, CPW, step, 0)
    plsc.subcore_barrier()
    pltpu.sync_copy(h_shared.at[pl.ds(tile_base, ROWS_PER_TILE)],
                    out_hbm.at[c, pl.ds(tile_base, ROWS_PER_TILE)])


def _combine_body(x_ref, lw_ref, bias_ref, parts_ref, out_ref):
    sl = lax.dot_general(x_ref[...], lw_ref[...], (((1,), (1,)), ((), ())),
                         preferred_element_type=jnp.float32)
    out_ref[...] = parts_ref[0] + parts_ref[1] + sl + bias_ref[0]


def kernel(x, edge_index, etypes, weight, w_comp, h_bias, loop_weight):
    src = edge_index[0]
    dst = edge_index[1]
    idx = etypes.astype(jnp.int32) * N + src.astype(jnp.int32)
    pad = E_PAD - E
    idx_p = jnp.concatenate([idx, jnp.zeros((pad,), jnp.int32)]).reshape(
        NCHUNKS, CHUNK)
    dst_p = jnp.concatenate([dst.astype(jnp.int32),
                             jnp.full((pad,), N, jnp.int32)]).reshape(
        NCHUNKS, CHUNK)

    xw = pl.pallas_call(
        _xw_body,
        grid=(N // BN, NUM_RELS),
        in_specs=[
            pl.BlockSpec(memory_space=pltpu.SMEM),
            pl.BlockSpec((NUM_BASES, IN_FEAT, OUT_FEAT), lambda nb, r: (0, 0, 0)),
            pl.BlockSpec((BN, IN_FEAT), lambda nb, r: (nb, 0)),
        ],
        out_specs=pl.BlockSpec((1, BN, OUT_FEAT), lambda nb, r: (r, nb, 0)),
        out_shape=jax.ShapeDtypeStruct((NUM_RELS, N, OUT_FEAT), jnp.float32),
    )(w_comp, weight, x)
    xw_flat = xw.reshape(NUM_RELS * N, OUT_FEAT)

    zeros_rows = jnp.zeros((ROWS_PER_TILE, OUT_FEAT), jnp.float32)

    mesh = plsc.VectorSubcoreMesh(core_axis_name="c", subcore_axis_name="s",
                                  num_cores=NC, num_subcores=NS)
    parts = pl.kernel(
        _sc_body,
        out_type=jax.ShapeDtypeStruct((NC, NPAD, OUT_FEAT), jnp.float32),
        mesh=mesh,
        scratch_types=[
            pltpu.VMEM((CHUNK,), jnp.int32),
            pltpu.VMEM((CHUNK,), jnp.int32),
            pltpu.VMEM((CHUNK, OUT_FEAT), jnp.float32),
            pltpu.VMEM_SHARED((NPAD, OUT_FEAT), jnp.float32),
            pltpu.SemaphoreType.DMA,
        ],
    )(xw_flat, idx_p, dst_p, zeros_rows)

    h = pl.pallas_call(
        _combine_body,
        grid=(N // BN,),
        in_specs=[
            pl.BlockSpec((BN, IN_FEAT), lambda nb: (nb, 0)),
            pl.BlockSpec((OUT_FEAT, IN_FEAT), lambda nb: (0, 0)),
            pl.BlockSpec((1, OUT_FEAT), lambda nb: (0, 0)),
            pl.BlockSpec((NC, BN, OUT_FEAT), lambda nb: (0, nb, 0)),
        ],
        out_specs=pl.BlockSpec((BN, OUT_FEAT), lambda nb: (nb, 0)),
        out_shape=jax.ShapeDtypeStruct((N, OUT_FEAT), jnp.float32),
    )(x, loop_weight, h_bias.reshape(1, OUT_FEAT), parts)
    return h
